# baseline (speedup 1.0000x reference)
import functools

import jax
import jax.numpy as jnp
from jax import lax
from jax.experimental import pallas as pl
from jax.experimental.pallas import tpu as pltpu
from jax.experimental.pallas import tpu_sc as plsc

Q, D, CAP, A, K_NN, H = 1024, 128, 100000, 8, 32, 64
KB = 2048
CAP_PAD = 100352
N_BLK = CAP_PAD // KB
CH = 256
NCH = CAP_PAD // CH
NCH_PAD = 512
NV = NCH_PAD // 16
NW = 32
QPW = Q // NW
NSEL = 48
NCAND = 96
NCV = NCAND // 16
BIG = 1e30
CUT = 1e29


def _stage_a(obs_ref, obs_sq_ref, keys_ref, key_sq_ref, w1_ref, b1_ref,
             w2_ref, b2_ref, w3_ref, b3_ref, dists_ref, cmin_ref, qnet_ref):
    i = pl.program_id(0)
    dot = lax.dot_general(obs_ref[...], keys_ref[...],
                          (((1,), (1,)), ((), ())),
                          preferred_element_type=jnp.float32)
    dblk = obs_sq_ref[...] - 2.0 * dot + key_sq_ref[...]
    dists_ref[...] = dblk.reshape(Q, KB // CH, CH)
    cmin_ref[...] = jnp.min(dblk.reshape(Q, KB // CH, CH), axis=2).reshape(
        1, Q, KB // CH)

    @pl.when(i == 0)
    def _mlp():
        h = jnp.maximum(
            lax.dot_general(obs_ref[...], w1_ref[...],
                            (((1,), (1,)), ((), ())),
                            preferred_element_type=jnp.float32) + b1_ref[...],
            0.0)
        h = jnp.maximum(
            lax.dot_general(h, w2_ref[...], (((1,), (1,)), ((), ())),
                            preferred_element_type=jnp.float32) + b2_ref[...],
            0.0)
        qnet_ref[...] = lax.dot_general(
            h, w3_ref[...], (((1,), (1,)), ((), ())),
            preferred_element_type=jnp.float32) + b3_ref[...]


def _splat_f(x):
    return jnp.full((16,), x, jnp.float32)


def _splat_i(x):
    return jnp.full((16,), x, jnp.int32)


_LANE15 = functools.partial(jnp.full, (16,), 15, jnp.int32)


_GDN = lax.GatherDimensionNumbers(
    offset_dims=(), collapsed_slice_dims=(0,), start_index_map=(0,))


def _lane_gather(v, idx):
    return lax.gather(v, idx[:, None], _GDN, (1,),
                      mode=lax.GatherScatterMode.PROMISE_IN_BOUNDS)


def _hmax(v):
    return _lane_gather(plsc.cummax(v), _LANE15())


def _hmin(v):
    return -_hmax(-v)


def _sc_body(dists_hbm, cmin_hbm, qnet_hbm, values_hbm, act_hbm,
             cmins_v, qnet_v, tbuf_v, chsel_v, chunks_v, cd_v, cp_v, sel_v,
             vrows_v, act_v, sem_c0, sem_c1, sem_v0, sem_v1):
    nc = 2
    wid = lax.axis_index("s") * nc + lax.axis_index("c")
    q0 = wid * QPW
    pltpu.sync_copy(cmin_hbm.at[pl.ds(q0, QPW)], cmins_v)
    pltpu.sync_copy(qnet_hbm.at[pl.ds(q0, QPW)], qnet_v)
    iota = lax.iota(jnp.int32, 16)
    sem_c = [sem_c0, sem_c1]
    sem_v = [sem_v0, sem_v1]

    def pass1(qi, carry):
        q = q0 + qi
        mn_a = cmins_v[qi, pl.ds(0, 16)]
        for j in range(1, NV // 2):
            mn_a = jnp.minimum(mn_a, cmins_v[qi, pl.ds(j * 16, 16)])
        mn_b = cmins_v[qi, pl.ds(NV // 2 * 16, 16)]
        for j in range(NV // 2 + 1, NV):
            mn_b = jnp.minimum(mn_b, cmins_v[qi, pl.ds(j * 16, 16)])
        lo0 = _hmin(jnp.minimum(mn_a, mn_b)) - 1.0
        hi0 = _hmax(jnp.maximum(mn_a, mn_b))

        def bis1(_, lohi):
            lo, hi = lohi
            mid = (lo + hi) * 0.5
            cnt = _splat_i(0)
            for j in range(NV):
                m = cmins_v[qi, pl.ds(j * 16, 16)] <= mid
                cnt = cnt + plsc.all_reduce_population_count(m)
            ge = cnt >= K_NN
            return jnp.where(ge, lo, mid), jnp.where(ge, mid, hi)

        _, t = lax.fori_loop(0, 16, bis1, (lo0, hi0))
        plsc.store_scatter(tbuf_v, [_splat_i(qi)], t, mask=iota == 0)

        padrow = _splat_i(q * NCH + (NCH - 1))
        for j in range(NSEL // 16):
            chsel_v[qi, pl.ds(j * 16, 16)] = padrow
        coff = _splat_i(0)
        for j in range(NV):
            v = cmins_v[qi, pl.ds(j * 16, 16)]
            m = v <= t
            pf = plsc.cumsum(jnp.where(m, 1, 0))
            pos = coff + pf - 1
            m = jnp.logical_and(m, pos < NSEL)
            rowid = _splat_i(q * NCH + j * 16) + iota
            plsc.store_scatter(chsel_v, [_splat_i(qi), pos], rowid, mask=m)
            coff = coff + plsc.all_reduce_population_count(m)
        return carry

    with jax.named_scope("sc_pass1"):
        lax.fori_loop(0, QPW, pass1, 0)

    pltpu.async_copy(dists_hbm.at[chsel_v.at[0]], chunks_v.at[0], sem_c[0])

    def p2a_one(qi, b):
        q = q0 + qi

        @pl.when(qi + 1 < QPW)
        def _prefetch():
            pltpu.async_copy(dists_hbm.at[chsel_v.at[qi + 1]],
                             chunks_v.at[1 - b], sem_c[1 - b])

        with jax.named_scope("sc_wait"):
            pltpu.make_async_copy(dists_hbm.at[chsel_v.at[qi]],
                                  chunks_v.at[b], sem_c[b]).wait()
        t = plsc.load_gather(tbuf_v, [_splat_i(qi)])
        for j in range(NCV):
            cd_v[pl.ds(j * 16, 16)] = _splat_f(BIG)

        def chunk_body(j, eoff):
            for e in range(CH // 16):
                v = chunks_v[b, j, pl.ds(e * 16, 16)]
                m = v <= t
                cnt = plsc.all_reduce_population_count(m)

                @pl.when(cnt[0] > 0)
                def _hit():
                    pf = plsc.cumsum(jnp.where(m, 1, 0))
                    pos = eoff + pf - 1
                    mm = jnp.logical_and(m, pos < NCAND)
                    plsc.store_scatter(cd_v, [pos], v, mask=mm)
                    plsc.store_scatter(cp_v, [pos],
                                       _splat_i(j * CH + e * 16) + iota,
                                       mask=mm)

                eoff = eoff + cnt
            return eoff

        with jax.named_scope("sc_filter"):
            lax.fori_loop(0, NSEL, chunk_body, _splat_i(0))

        cmn = cd_v[pl.ds(0, 16)]
        for j in range(1, NCV):
            cmn = jnp.minimum(cmn, cd_v[pl.ds(j * 16, 16)])
        clo0 = _hmin(cmn) - 1.0

        def bis2(_, lohi):
            lo2, hi2 = lohi
            mid = (lo2 + hi2) * 0.5
            cnt = _splat_i(0)
            for j in range(NCV):
                m = cd_v[pl.ds(j * 16, 16)] <= mid
                cnt = cnt + plsc.all_reduce_population_count(m)
            ge = cnt >= K_NN
            return jnp.where(ge, lo2, mid), jnp.where(ge, mid, hi2)

        with jax.named_scope("sc_bis2"):
            clo, chi = lax.fori_loop(0, 40, bis2, (clo0, t))

        c1 = _splat_i(0)
        for j in range(NCV):
            c1 = c1 + plsc.all_reduce_population_count(
                cd_v[pl.ds(j * 16, 16)] <= clo)
        need = _splat_i(K_NN) - c1
        c2run = _splat_i(0)
        soff = _splat_i(0)
        for j in range(NCV):
            d = cd_v[pl.ds(j * 16, 16)]
            p = cp_v[pl.ds(j * 16, 16)]
            m1 = d <= clo
            m2 = jnp.logical_and(d <= chi, jnp.logical_not(m1))
            pf2 = plsc.cumsum(jnp.where(m2, 1, 0))
            sel2 = jnp.logical_and(m2, (pf2 + c2run) <= need)
            c2run = c2run + plsc.all_reduce_population_count(m2)
            selm = jnp.logical_or(m1, sel2)
            rowv = plsc.load_gather(
                chsel_v.at[qi], [lax.shift_right_logical(p, 8)], mask=selm)
            gkey = (rowv - q * NCH) * CH + jnp.bitwise_and(p, 255)
            pfs = plsc.cumsum(jnp.where(selm, 1, 0))
            plsc.store_scatter(sel_v, [_splat_i(qi), soff + pfs - 1], gkey,
                               mask=selm)
            soff = soff + plsc.all_reduce_population_count(selm)

    def pass2a(i, carry):
        p2a_one(2 * i, 0)
        p2a_one(2 * i + 1, 1)
        return carry

    with jax.named_scope("sc_pass2a"):
        lax.fori_loop(0, QPW // 2, pass2a, 0)

    pltpu.async_copy(values_hbm.at[sel_v.at[0]], vrows_v.at[0], sem_v[0])

    def p2b_one(qi, b):
        @pl.when(qi + 1 < QPW)
        def _prefetch():
            pltpu.async_copy(values_hbm.at[sel_v.at[qi + 1]],
                             vrows_v.at[1 - b], sem_v[1 - b])

        pltpu.make_async_copy(values_hbm.at[sel_v.at[qi]],
                              vrows_v.at[b], sem_v[b]).wait()
        acc = vrows_v[b, 0, pl.ds(0, 16)]
        for r in range(1, K_NN):
            acc = acc + vrows_v[b, r, pl.ds(0, 16)]
        qvec = acc * (1.0 / K_NN) + qnet_v[qi, pl.ds(0, 16)]
        amax = _hmax(qvec)
        aidx = jnp.where(qvec == amax, iota, 16)
        act = -_hmax(-aidx)
        plsc.store_scatter(act_v, [_splat_i(qi)], act, mask=iota == 0)

    def pass2b(i, carry):
        p2b_one(2 * i, 0)
        p2b_one(2 * i + 1, 1)
        return carry

    with jax.named_scope("sc_pass2b"):
        lax.fori_loop(0, QPW // 2, pass2b, 0)
    pltpu.sync_copy(act_v, act_hbm.at[pl.ds(q0, QPW)])


def _make_sc_kernel():
    return functools.partial(
        pl.kernel,
        out_type=jax.ShapeDtypeStruct((Q,), jnp.int32),
        mesh=plsc.VectorSubcoreMesh(core_axis_name="c", subcore_axis_name="s",
                                    num_cores=2, num_subcores=16),
        compiler_params=pltpu.CompilerParams(needs_layout_passes=False,
                                             use_tc_tiling_on_sc=True),
        scratch_types=[
            pltpu.VMEM((QPW, NCH_PAD), jnp.float32),
            pltpu.VMEM((QPW, 128), jnp.float32),
            pltpu.VMEM((QPW,), jnp.float32),
            pltpu.VMEM((QPW, NSEL), jnp.int32),
            pltpu.VMEM((2, NSEL, CH), jnp.float32),
            pltpu.VMEM((NCAND,), jnp.float32),
            pltpu.VMEM((NCAND,), jnp.int32),
            pltpu.VMEM((QPW, K_NN), jnp.int32),
            pltpu.VMEM((2, K_NN, 128), jnp.float32),
            pltpu.VMEM((QPW,), jnp.int32),
            pltpu.SemaphoreType.DMA,
            pltpu.SemaphoreType.DMA,
            pltpu.SemaphoreType.DMA,
            pltpu.SemaphoreType.DMA,
        ],
    )(_sc_body)


def kernel(observation, keys, values, W1, b1, W2, b2, W3, b3):
    pad = CAP_PAD - CAP
    keys_p = jnp.concatenate([keys, jnp.zeros((pad, D), jnp.float32)], axis=0)
    key_sq = jnp.sum(keys * keys, axis=-1)
    key_sq_p = jnp.concatenate([key_sq, jnp.full((pad,), BIG, jnp.float32)])
    obs_sq = jnp.sum(observation * observation, axis=-1, keepdims=True)
    w3_p = jnp.concatenate([W3, jnp.zeros((128 - A, H), jnp.float32)], axis=0)
    b3_p = jnp.concatenate([b3, jnp.full((128 - A,), -BIG, jnp.float32)])
    values_p = jnp.concatenate(
        [values, jnp.zeros((CAP, 128 - A), jnp.float32)], axis=1)

    full = lambda s: pl.BlockSpec(s, lambda i: tuple(0 for _ in s))
    dists, cmin3, qnet = pl.pallas_call(
        _stage_a,
        grid=(N_BLK,),
        in_specs=[
            full((Q, D)),
            full((Q, 1)),
            pl.BlockSpec((KB, D), lambda i: (i, 0)),
            pl.BlockSpec((1, KB), lambda i: (0, i)),
            full((H, D)),
            full((1, H)),
            full((H, H)),
            full((1, H)),
            full((128, H)),
            full((1, 128)),
        ],
        out_specs=[
            pl.BlockSpec((Q, KB // CH, CH), lambda i: (0, i, 0)),
            pl.BlockSpec((1, Q, KB // CH), lambda i: (i, 0, 0)),
            full((Q, 128)),
        ],
        out_shape=[
            jax.ShapeDtypeStruct((Q, NCH, CH), jnp.float32),
            jax.ShapeDtypeStruct((N_BLK, Q, KB // CH), jnp.float32),
            jax.ShapeDtypeStruct((Q, 128), jnp.float32),
        ],
    )(observation, obs_sq, keys_p, key_sq_p.reshape(1, CAP_PAD),
      W1, b1.reshape(1, H), W2, b2.reshape(1, H), w3_p, b3_p.reshape(1, 128))

    cmin = jnp.transpose(cmin3, (1, 0, 2)).reshape(Q, NCH)
    cmin = jnp.concatenate(
        [cmin, jnp.full((Q, NCH_PAD - NCH), BIG, jnp.float32)], axis=1)
    dists_flat = dists.reshape(Q * NCH, CH)

    return _make_sc_kernel()(dists_flat, cmin, qnet, values_p)


# speedup vs baseline: 1.4602x; 1.4602x over previous
import functools

import jax
import jax.numpy as jnp
from jax import lax
from jax.experimental import pallas as pl
from jax.experimental.pallas import tpu as pltpu
from jax.experimental.pallas import tpu_sc as plsc

Q, D, CAP, A, K_NN, H = 1024, 128, 100000, 8, 32, 64
KB = 2048
CAP_PAD = 100352
N_BLK = CAP_PAD // KB
CH = 256
NCH = CAP_PAD // CH
NCH_PAD = 512
NV = NCH_PAD // 16
NW = 32
QPW = Q // NW
NSEL = 48
NCAND = 96
NCV = NCAND // 16
BIG = 1e30
CUT = 1e29


def _stage_a(obs_ref, obs_sq_ref, keys_ref, key_sq_ref, w1_ref, b1_ref,
             w2_ref, b2_ref, w3_ref, b3_ref, dists_ref, cmin_ref, qnet_ref):
    i = pl.program_id(0)
    dot = lax.dot_general(obs_ref[...], keys_ref[...],
                          (((1,), (1,)), ((), ())),
                          preferred_element_type=jnp.float32)
    dblk = obs_sq_ref[...] - 2.0 * dot + key_sq_ref[...]
    dists_ref[...] = dblk.reshape(Q, KB // CH, CH)
    cmin_ref[...] = jnp.min(dblk.reshape(Q, KB // CH, CH), axis=2).reshape(
        1, Q, KB // CH)

    @pl.when(i == 0)
    def _mlp():
        h = jnp.maximum(
            lax.dot_general(obs_ref[...], w1_ref[...],
                            (((1,), (1,)), ((), ())),
                            preferred_element_type=jnp.float32) + b1_ref[...],
            0.0)
        h = jnp.maximum(
            lax.dot_general(h, w2_ref[...], (((1,), (1,)), ((), ())),
                            preferred_element_type=jnp.float32) + b2_ref[...],
            0.0)
        qnet_ref[...] = lax.dot_general(
            h, w3_ref[...], (((1,), (1,)), ((), ())),
            preferred_element_type=jnp.float32) + b3_ref[...]


def _splat_f(x):
    return jnp.full((16,), x, jnp.float32)


def _splat_i(x):
    return jnp.full((16,), x, jnp.int32)


_LANE15 = functools.partial(jnp.full, (16,), 15, jnp.int32)


_GDN = lax.GatherDimensionNumbers(
    offset_dims=(), collapsed_slice_dims=(0,), start_index_map=(0,))


def _lane_gather(v, idx):
    return lax.gather(v, idx[:, None], _GDN, (1,),
                      mode=lax.GatherScatterMode.PROMISE_IN_BOUNDS)


def _hmax(v):
    return _lane_gather(plsc.cummax(v), _LANE15())


def _hmin(v):
    return -_hmax(-v)


def _sc_body(dists_hbm, cmin_hbm, qnet_hbm, values_hbm, act_hbm,
             cmins_v, qnet_v, tbuf_v, chsel_v, chunks_v, cd_v, cp_v, sel_v,
             vrows_v, act_v, sem_c0, sem_c1, sem_v0, sem_v1):
    nc = 2
    wid = lax.axis_index("s") * nc + lax.axis_index("c")
    q0 = wid * QPW
    pltpu.sync_copy(cmin_hbm.at[pl.ds(q0, QPW)], cmins_v)
    pltpu.sync_copy(qnet_hbm.at[pl.ds(q0, QPW)], qnet_v)
    iota = lax.iota(jnp.int32, 16)
    sem_c = [sem_c0, sem_c1]
    sem_v = [sem_v0, sem_v1]

    def pass1(qi, carry):
        q = q0 + qi
        mn_a = cmins_v[qi, pl.ds(0, 16)]
        for j in range(1, NV // 2):
            mn_a = jnp.minimum(mn_a, cmins_v[qi, pl.ds(j * 16, 16)])
        mn_b = cmins_v[qi, pl.ds(NV // 2 * 16, 16)]
        for j in range(NV // 2 + 1, NV):
            mn_b = jnp.minimum(mn_b, cmins_v[qi, pl.ds(j * 16, 16)])
        lo0 = _hmin(jnp.minimum(mn_a, mn_b)) - 1.0
        hi0 = _hmax(jnp.maximum(mn_a, mn_b))

        def bis1(_, lohi):
            lo, hi = lohi
            mid = (lo + hi) * 0.5
            cnt = _splat_i(0)
            for j in range(NV):
                m = cmins_v[qi, pl.ds(j * 16, 16)] <= mid
                cnt = cnt + plsc.all_reduce_population_count(m)
            ge = cnt >= K_NN
            return jnp.where(ge, lo, mid), jnp.where(ge, mid, hi)

        _, t = lax.fori_loop(0, 16, bis1, (lo0, hi0))
        plsc.store_scatter(tbuf_v, [_splat_i(qi)], t, mask=iota == 0)

        padrow = _splat_i(q * NCH + (NCH - 1))
        for j in range(NSEL // 16):
            chsel_v[qi, pl.ds(j * 16, 16)] = padrow
        coff = _splat_i(0)
        for j in range(NV):
            v = cmins_v[qi, pl.ds(j * 16, 16)]
            m = v <= t
            pf = plsc.cumsum(jnp.where(m, 1, 0))
            pos = coff + pf - 1
            m = jnp.logical_and(m, pos < NSEL)
            rowid = _splat_i(q * NCH + j * 16) + iota
            plsc.store_scatter(chsel_v, [_splat_i(qi), pos], rowid, mask=m)
            coff = coff + plsc.all_reduce_population_count(m)
        return carry

    with jax.named_scope("sc_pass1"):
        lax.fori_loop(0, QPW, pass1, 0)

    pltpu.async_copy(dists_hbm.at[chsel_v.at[0]], chunks_v.at[0], sem_c[0])

    def p2a_one(qi, b):
        q = q0 + qi

        @pl.when(qi + 1 < QPW)
        def _prefetch():
            pltpu.async_copy(dists_hbm.at[chsel_v.at[qi + 1]],
                             chunks_v.at[1 - b], sem_c[1 - b])

        with jax.named_scope("sc_wait"):
            pltpu.make_async_copy(dists_hbm.at[chsel_v.at[qi]],
                                  chunks_v.at[b], sem_c[b]).wait()
        t = plsc.load_gather(tbuf_v, [_splat_i(qi)])
        for j in range(NCV):
            cd_v[pl.ds(j * 16, 16)] = _splat_f(BIG)

        def chunk_body(j, eoff):
            for e in range(CH // 16):
                v = chunks_v[b, j, pl.ds(e * 16, 16)]
                cnt = plsc.all_reduce_population_count(v <= t)
                lpos = _splat_i(j * CH + e * 16) + iota
                sv, sp = plsc.sort_key_val(v, lpos)
                pos = jnp.minimum(eoff + iota, NCAND - 1)
                plsc.store_scatter(cd_v, [pos], sv)
                plsc.store_scatter(cp_v, [pos], sp)
                eoff = eoff + cnt
            return eoff

        with jax.named_scope("sc_filter"):
            lax.fori_loop(0, NSEL, chunk_body, _splat_i(0))

        cmn = cd_v[pl.ds(0, 16)]
        for j in range(1, NCV):
            cmn = jnp.minimum(cmn, cd_v[pl.ds(j * 16, 16)])
        clo0 = _hmin(cmn) - 1.0

        def bis2(_, lohi):
            lo2, hi2 = lohi
            mid = (lo2 + hi2) * 0.5
            cnt = _splat_i(0)
            for j in range(NCV):
                m = cd_v[pl.ds(j * 16, 16)] <= mid
                cnt = cnt + plsc.all_reduce_population_count(m)
            ge = cnt >= K_NN
            return jnp.where(ge, lo2, mid), jnp.where(ge, mid, hi2)

        with jax.named_scope("sc_bis2"):
            clo, chi = lax.fori_loop(0, 40, bis2, (clo0, t))

        c1 = _splat_i(0)
        for j in range(NCV):
            c1 = c1 + plsc.all_reduce_population_count(
                cd_v[pl.ds(j * 16, 16)] <= clo)
        need = _splat_i(K_NN) - c1
        c2run = _splat_i(0)
        soff = _splat_i(0)
        for j in range(NCV):
            d = cd_v[pl.ds(j * 16, 16)]
            p = cp_v[pl.ds(j * 16, 16)]
            m1 = d <= clo
            m2 = jnp.logical_and(d <= chi, jnp.logical_not(m1))
            pf2 = plsc.cumsum(jnp.where(m2, 1, 0))
            sel2 = jnp.logical_and(m2, (pf2 + c2run) <= need)
            c2run = c2run + plsc.all_reduce_population_count(m2)
            selm = jnp.logical_or(m1, sel2)
            rowv = plsc.load_gather(
                chsel_v.at[qi], [lax.shift_right_logical(p, 8)], mask=selm)
            gkey = (rowv - q * NCH) * CH + jnp.bitwise_and(p, 255)
            pfs = plsc.cumsum(jnp.where(selm, 1, 0))
            plsc.store_scatter(sel_v, [_splat_i(qi), soff + pfs - 1], gkey,
                               mask=selm)
            soff = soff + plsc.all_reduce_population_count(selm)

    def pass2a(i, carry):
        p2a_one(2 * i, 0)
        p2a_one(2 * i + 1, 1)
        return carry

    with jax.named_scope("sc_pass2a"):
        lax.fori_loop(0, QPW // 2, pass2a, 0)

    pltpu.async_copy(values_hbm.at[sel_v.at[0]], vrows_v.at[0], sem_v[0])

    def p2b_one(qi, b):
        @pl.when(qi + 1 < QPW)
        def _prefetch():
            pltpu.async_copy(values_hbm.at[sel_v.at[qi + 1]],
                             vrows_v.at[1 - b], sem_v[1 - b])

        pltpu.make_async_copy(values_hbm.at[sel_v.at[qi]],
                              vrows_v.at[b], sem_v[b]).wait()
        acc = vrows_v[b, 0, pl.ds(0, 16)]
        for r in range(1, K_NN):
            acc = acc + vrows_v[b, r, pl.ds(0, 16)]
        qvec = acc * (1.0 / K_NN) + qnet_v[qi, pl.ds(0, 16)]
        amax = _hmax(qvec)
        aidx = jnp.where(qvec == amax, iota, 16)
        act = -_hmax(-aidx)
        plsc.store_scatter(act_v, [_splat_i(qi)], act, mask=iota == 0)

    def pass2b(i, carry):
        p2b_one(2 * i, 0)
        p2b_one(2 * i + 1, 1)
        return carry

    with jax.named_scope("sc_pass2b"):
        lax.fori_loop(0, QPW // 2, pass2b, 0)
    pltpu.sync_copy(act_v, act_hbm.at[pl.ds(q0, QPW)])


def _make_sc_kernel():
    return functools.partial(
        pl.kernel,
        out_type=jax.ShapeDtypeStruct((Q,), jnp.int32),
        mesh=plsc.VectorSubcoreMesh(core_axis_name="c", subcore_axis_name="s",
                                    num_cores=2, num_subcores=16),
        compiler_params=pltpu.CompilerParams(needs_layout_passes=False,
                                             use_tc_tiling_on_sc=True),
        scratch_types=[
            pltpu.VMEM((QPW, NCH_PAD), jnp.float32),
            pltpu.VMEM((QPW, 128), jnp.float32),
            pltpu.VMEM((QPW,), jnp.float32),
            pltpu.VMEM((QPW, NSEL), jnp.int32),
            pltpu.VMEM((2, NSEL, CH), jnp.float32),
            pltpu.VMEM((NCAND,), jnp.float32),
            pltpu.VMEM((NCAND,), jnp.int32),
            pltpu.VMEM((QPW, K_NN), jnp.int32),
            pltpu.VMEM((2, K_NN, 128), jnp.float32),
            pltpu.VMEM((QPW,), jnp.int32),
            pltpu.SemaphoreType.DMA,
            pltpu.SemaphoreType.DMA,
            pltpu.SemaphoreType.DMA,
            pltpu.SemaphoreType.DMA,
        ],
    )(_sc_body)


def kernel(observation, keys, values, W1, b1, W2, b2, W3, b3):
    pad = CAP_PAD - CAP
    keys_p = jnp.concatenate([keys, jnp.zeros((pad, D), jnp.float32)], axis=0)
    key_sq = jnp.sum(keys * keys, axis=-1)
    key_sq_p = jnp.concatenate([key_sq, jnp.full((pad,), BIG, jnp.float32)])
    obs_sq = jnp.sum(observation * observation, axis=-1, keepdims=True)
    w3_p = jnp.concatenate([W3, jnp.zeros((128 - A, H), jnp.float32)], axis=0)
    b3_p = jnp.concatenate([b3, jnp.full((128 - A,), -BIG, jnp.float32)])
    values_p = jnp.concatenate(
        [values, jnp.zeros((CAP, 128 - A), jnp.float32)], axis=1)

    full = lambda s: pl.BlockSpec(s, lambda i: tuple(0 for _ in s))
    dists, cmin3, qnet = pl.pallas_call(
        _stage_a,
        grid=(N_BLK,),
        in_specs=[
            full((Q, D)),
            full((Q, 1)),
            pl.BlockSpec((KB, D), lambda i: (i, 0)),
            pl.BlockSpec((1, KB), lambda i: (0, i)),
            full((H, D)),
            full((1, H)),
            full((H, H)),
            full((1, H)),
            full((128, H)),
            full((1, 128)),
        ],
        out_specs=[
            pl.BlockSpec((Q, KB // CH, CH), lambda i: (0, i, 0)),
            pl.BlockSpec((1, Q, KB // CH), lambda i: (i, 0, 0)),
            full((Q, 128)),
        ],
        out_shape=[
            jax.ShapeDtypeStruct((Q, NCH, CH), jnp.float32),
            jax.ShapeDtypeStruct((N_BLK, Q, KB // CH), jnp.float32),
            jax.ShapeDtypeStruct((Q, 128), jnp.float32),
        ],
    )(observation, obs_sq, keys_p, key_sq_p.reshape(1, CAP_PAD),
      W1, b1.reshape(1, H), W2, b2.reshape(1, H), w3_p, b3_p.reshape(1, 128))

    cmin = jnp.transpose(cmin3, (1, 0, 2)).reshape(Q, NCH)
    cmin = jnp.concatenate(
        [cmin, jnp.full((Q, NCH_PAD - NCH), BIG, jnp.float32)], axis=1)
    dists_flat = dists.reshape(Q * NCH, CH)

    return _make_sc_kernel()(dists_flat, cmin, qnet, values_p)


# speedup vs baseline: 1.7146x; 1.1742x over previous
import functools

import jax
import jax.numpy as jnp
from jax import lax
from jax.experimental import pallas as pl
from jax.experimental.pallas import tpu as pltpu
from jax.experimental.pallas import tpu_sc as plsc

Q, D, CAP, A, K_NN, H = 1024, 128, 100000, 8, 32, 64
KB = 2048
CAP_PAD = 100352
N_BLK = CAP_PAD // KB
CH = 128
NCH = CAP_PAD // CH
NCH_PAD = 896
NV = NCH_PAD // 16
NW = 32
QPW = Q // NW
NSEL = 48
NCAND = 96
NCV = NCAND // 16
BIG = 1e30
CUT = 1e29


def _stage_a(obs_ref, obs_sq_ref, keys_ref, key_sq_ref, w1_ref, b1_ref,
             w2_ref, b2_ref, w3_ref, b3_ref, dists_ref, cmin_ref, qnet_ref):
    i = pl.program_id(0)
    dot = lax.dot_general(obs_ref[...], keys_ref[...],
                          (((1,), (1,)), ((), ())),
                          preferred_element_type=jnp.float32)
    dblk = obs_sq_ref[...] - 2.0 * dot + key_sq_ref[...]
    dists_ref[...] = dblk.reshape(Q, KB // CH, CH)
    cmin_ref[...] = jnp.min(dblk.reshape(Q, KB // CH, CH), axis=2).reshape(
        1, Q, KB // CH)

    @pl.when(i == 0)
    def _mlp():
        h = jnp.maximum(
            lax.dot_general(obs_ref[...], w1_ref[...],
                            (((1,), (1,)), ((), ())),
                            preferred_element_type=jnp.float32) + b1_ref[...],
            0.0)
        h = jnp.maximum(
            lax.dot_general(h, w2_ref[...], (((1,), (1,)), ((), ())),
                            preferred_element_type=jnp.float32) + b2_ref[...],
            0.0)
        qnet_ref[...] = lax.dot_general(
            h, w3_ref[...], (((1,), (1,)), ((), ())),
            preferred_element_type=jnp.float32) + b3_ref[...]


def _splat_f(x):
    return jnp.full((16,), x, jnp.float32)


def _splat_i(x):
    return jnp.full((16,), x, jnp.int32)


_LANE15 = functools.partial(jnp.full, (16,), 15, jnp.int32)


_GDN = lax.GatherDimensionNumbers(
    offset_dims=(), collapsed_slice_dims=(0,), start_index_map=(0,))


def _lane_gather(v, idx):
    return lax.gather(v, idx[:, None], _GDN, (1,),
                      mode=lax.GatherScatterMode.PROMISE_IN_BOUNDS)


def _hmax(v):
    return _lane_gather(plsc.cummax(v), _LANE15())


def _hmin(v):
    return -_hmax(-v)


def _sc_body(dists_hbm, cmin_hbm, qnet_hbm, values_hbm, act_hbm,
             cmins_v, qnet_v, tbuf_v, chsel_v, chunks_v, cd_v, cp_v, sel_v,
             vrows_v, act_v, sem_c0, sem_c1, sem_v0, sem_v1):
    nc = 2
    wid = lax.axis_index("s") * nc + lax.axis_index("c")
    q0 = wid * QPW
    pltpu.sync_copy(cmin_hbm.at[pl.ds(q0, QPW)], cmins_v)
    pltpu.sync_copy(qnet_hbm.at[pl.ds(q0, QPW)], qnet_v)
    iota = lax.iota(jnp.int32, 16)
    sem_c = [sem_c0, sem_c1]
    sem_v = [sem_v0, sem_v1]

    def pass1(qi, carry):
        q = q0 + qi
        mn_a = cmins_v[qi, pl.ds(0, 16)]
        for j in range(1, NV // 2):
            mn_a = jnp.minimum(mn_a, cmins_v[qi, pl.ds(j * 16, 16)])
        mn_b = cmins_v[qi, pl.ds(NV // 2 * 16, 16)]
        for j in range(NV // 2 + 1, NV):
            mn_b = jnp.minimum(mn_b, cmins_v[qi, pl.ds(j * 16, 16)])
        lo0 = _hmin(jnp.minimum(mn_a, mn_b)) - 1.0
        hi0 = _hmax(jnp.maximum(mn_a, mn_b))

        def bis1(_, lohi):
            lo, hi = lohi
            mid = (lo + hi) * 0.5
            cnt = _splat_i(0)
            for j in range(NV):
                m = cmins_v[qi, pl.ds(j * 16, 16)] <= mid
                cnt = cnt + plsc.all_reduce_population_count(m)
            ge = cnt >= K_NN
            return jnp.where(ge, lo, mid), jnp.where(ge, mid, hi)

        _, t = lax.fori_loop(0, 16, bis1, (lo0, hi0))
        plsc.store_scatter(tbuf_v, [_splat_i(qi)], t, mask=iota == 0)

        padrow = _splat_i(q * NCH + (NCH - 1))
        for j in range(NSEL // 16):
            chsel_v[qi, pl.ds(j * 16, 16)] = padrow
        coff = _splat_i(0)
        for j in range(NV):
            v = cmins_v[qi, pl.ds(j * 16, 16)]
            m = v <= t
            pf = plsc.cumsum(jnp.where(m, 1, 0))
            pos = coff + pf - 1
            m = jnp.logical_and(m, pos < NSEL)
            rowid = _splat_i(q * NCH + j * 16) + iota
            plsc.store_scatter(chsel_v, [_splat_i(qi), pos], rowid, mask=m)
            coff = coff + plsc.all_reduce_population_count(m)
        return carry

    with jax.named_scope("sc_pass1"):
        lax.fori_loop(0, QPW, pass1, 0)

    pltpu.async_copy(dists_hbm.at[chsel_v.at[0]], chunks_v.at[0], sem_c[0])

    def p2a_one(qi, b):
        q = q0 + qi

        @pl.when(qi + 1 < QPW)
        def _prefetch():
            pltpu.async_copy(dists_hbm.at[chsel_v.at[qi + 1]],
                             chunks_v.at[1 - b], sem_c[1 - b])

        with jax.named_scope("sc_wait"):
            pltpu.make_async_copy(dists_hbm.at[chsel_v.at[qi]],
                                  chunks_v.at[b], sem_c[b]).wait()
        t = plsc.load_gather(tbuf_v, [_splat_i(qi)])
        for j in range(NCV):
            cd_v[pl.ds(j * 16, 16)] = _splat_f(BIG)

        def chunk_body(j, eoff):
            for e in range(CH // 16):
                v = chunks_v[b, j, pl.ds(e * 16, 16)]
                cnt = plsc.all_reduce_population_count(v <= t)
                lpos = _splat_i(j * CH + e * 16) + iota
                sv, sp = plsc.sort_key_val(v, lpos)
                pos = jnp.minimum(eoff + iota, NCAND - 1)
                plsc.store_scatter(cd_v, [pos], sv)
                plsc.store_scatter(cp_v, [pos], sp)
                eoff = eoff + cnt
            return eoff

        with jax.named_scope("sc_filter"):
            lax.fori_loop(0, NSEL, chunk_body, _splat_i(0))

        cmn = cd_v[pl.ds(0, 16)]
        for j in range(1, NCV):
            cmn = jnp.minimum(cmn, cd_v[pl.ds(j * 16, 16)])
        clo0 = _hmin(cmn) - 1.0

        def bis2(_, lohi):
            lo2, hi2 = lohi
            mid = (lo2 + hi2) * 0.5
            cnt = _splat_i(0)
            for j in range(NCV):
                m = cd_v[pl.ds(j * 16, 16)] <= mid
                cnt = cnt + plsc.all_reduce_population_count(m)
            ge = cnt >= K_NN
            return jnp.where(ge, lo2, mid), jnp.where(ge, mid, hi2)

        with jax.named_scope("sc_bis2"):
            clo, chi = lax.fori_loop(0, 40, bis2, (clo0, t))

        c1 = _splat_i(0)
        for j in range(NCV):
            c1 = c1 + plsc.all_reduce_population_count(
                cd_v[pl.ds(j * 16, 16)] <= clo)
        need = _splat_i(K_NN) - c1
        c2run = _splat_i(0)
        soff = _splat_i(0)
        for j in range(NCV):
            d = cd_v[pl.ds(j * 16, 16)]
            p = cp_v[pl.ds(j * 16, 16)]
            m1 = d <= clo
            m2 = jnp.logical_and(d <= chi, jnp.logical_not(m1))
            pf2 = plsc.cumsum(jnp.where(m2, 1, 0))
            sel2 = jnp.logical_and(m2, (pf2 + c2run) <= need)
            c2run = c2run + plsc.all_reduce_population_count(m2)
            selm = jnp.logical_or(m1, sel2)
            rowv = plsc.load_gather(
                chsel_v.at[qi], [lax.shift_right_logical(p, 7)], mask=selm)
            gkey = (rowv - q * NCH) * CH + jnp.bitwise_and(p, CH - 1)
            pfs = plsc.cumsum(jnp.where(selm, 1, 0))
            plsc.store_scatter(sel_v, [_splat_i(qi), soff + pfs - 1], gkey,
                               mask=selm)
            soff = soff + plsc.all_reduce_population_count(selm)

    def pass2a(i, carry):
        p2a_one(2 * i, 0)
        p2a_one(2 * i + 1, 1)
        return carry

    with jax.named_scope("sc_pass2a"):
        lax.fori_loop(0, QPW // 2, pass2a, 0)

    pltpu.async_copy(values_hbm.at[sel_v.at[0]], vrows_v.at[0], sem_v[0])

    def p2b_one(qi, b):
        @pl.when(qi + 1 < QPW)
        def _prefetch():
            pltpu.async_copy(values_hbm.at[sel_v.at[qi + 1]],
                             vrows_v.at[1 - b], sem_v[1 - b])

        pltpu.make_async_copy(values_hbm.at[sel_v.at[qi]],
                              vrows_v.at[b], sem_v[b]).wait()
        acc = vrows_v[b, 0, pl.ds(0, 16)]
        for r in range(1, K_NN):
            acc = acc + vrows_v[b, r, pl.ds(0, 16)]
        qvec = acc * (1.0 / K_NN) + qnet_v[qi, pl.ds(0, 16)]
        amax = _hmax(qvec)
        aidx = jnp.where(qvec == amax, iota, 16)
        act = -_hmax(-aidx)
        plsc.store_scatter(act_v, [_splat_i(qi)], act, mask=iota == 0)

    def pass2b(i, carry):
        p2b_one(2 * i, 0)
        p2b_one(2 * i + 1, 1)
        return carry

    with jax.named_scope("sc_pass2b"):
        lax.fori_loop(0, QPW // 2, pass2b, 0)
    pltpu.sync_copy(act_v, act_hbm.at[pl.ds(q0, QPW)])


def _make_sc_kernel():
    return functools.partial(
        pl.kernel,
        out_type=jax.ShapeDtypeStruct((Q,), jnp.int32),
        mesh=plsc.VectorSubcoreMesh(core_axis_name="c", subcore_axis_name="s",
                                    num_cores=2, num_subcores=16),
        compiler_params=pltpu.CompilerParams(needs_layout_passes=False,
                                             use_tc_tiling_on_sc=True),
        scratch_types=[
            pltpu.VMEM((QPW, NCH_PAD), jnp.float32),
            pltpu.VMEM((QPW, 128), jnp.float32),
            pltpu.VMEM((QPW,), jnp.float32),
            pltpu.VMEM((QPW, NSEL), jnp.int32),
            pltpu.VMEM((2, NSEL, CH), jnp.float32),
            pltpu.VMEM((NCAND,), jnp.float32),
            pltpu.VMEM((NCAND,), jnp.int32),
            pltpu.VMEM((QPW, K_NN), jnp.int32),
            pltpu.VMEM((2, K_NN, 128), jnp.float32),
            pltpu.VMEM((QPW,), jnp.int32),
            pltpu.SemaphoreType.DMA,
            pltpu.SemaphoreType.DMA,
            pltpu.SemaphoreType.DMA,
            pltpu.SemaphoreType.DMA,
        ],
    )(_sc_body)


def kernel(observation, keys, values, W1, b1, W2, b2, W3, b3):
    pad = CAP_PAD - CAP
    keys_p = jnp.concatenate([keys, jnp.zeros((pad, D), jnp.float32)], axis=0)
    key_sq = jnp.sum(keys * keys, axis=-1)
    key_sq_p = jnp.concatenate([key_sq, jnp.full((pad,), BIG, jnp.float32)])
    obs_sq = jnp.sum(observation * observation, axis=-1, keepdims=True)
    w3_p = jnp.concatenate([W3, jnp.zeros((128 - A, H), jnp.float32)], axis=0)
    b3_p = jnp.concatenate([b3, jnp.full((128 - A,), -BIG, jnp.float32)])
    values_p = jnp.concatenate(
        [values, jnp.zeros((CAP, 128 - A), jnp.float32)], axis=1)

    full = lambda s: pl.BlockSpec(s, lambda i: tuple(0 for _ in s))
    dists, cmin3, qnet = pl.pallas_call(
        _stage_a,
        grid=(N_BLK,),
        in_specs=[
            full((Q, D)),
            full((Q, 1)),
            pl.BlockSpec((KB, D), lambda i: (i, 0)),
            pl.BlockSpec((1, KB), lambda i: (0, i)),
            full((H, D)),
            full((1, H)),
            full((H, H)),
            full((1, H)),
            full((128, H)),
            full((1, 128)),
        ],
        out_specs=[
            pl.BlockSpec((Q, KB // CH, CH), lambda i: (0, i, 0)),
            pl.BlockSpec((1, Q, KB // CH), lambda i: (i, 0, 0)),
            full((Q, 128)),
        ],
        out_shape=[
            jax.ShapeDtypeStruct((Q, NCH, CH), jnp.float32),
            jax.ShapeDtypeStruct((N_BLK, Q, KB // CH), jnp.float32),
            jax.ShapeDtypeStruct((Q, 128), jnp.float32),
        ],
    )(observation, obs_sq, keys_p, key_sq_p.reshape(1, CAP_PAD),
      W1, b1.reshape(1, H), W2, b2.reshape(1, H), w3_p, b3_p.reshape(1, 128))

    cmin = jnp.transpose(cmin3, (1, 0, 2)).reshape(Q, NCH)
    cmin = jnp.concatenate(
        [cmin, jnp.full((Q, NCH_PAD - NCH), BIG, jnp.float32)], axis=1)
    dists_flat = dists.reshape(Q * NCH, CH)

    return _make_sc_kernel()(dists_flat, cmin, qnet, values_p)


# speedup vs baseline: 1.7361x; 1.0126x over previous
import functools

import jax
import jax.numpy as jnp
from jax import lax
from jax.experimental import pallas as pl
from jax.experimental.pallas import tpu as pltpu
from jax.experimental.pallas import tpu_sc as plsc

Q, D, CAP, A, K_NN, H = 1024, 128, 100000, 8, 32, 64
KB = 2048
CAP_PAD = 100352
N_BLK = CAP_PAD // KB
CH = 128
NCH = CAP_PAD // CH
NCH_PAD = 896
NV = NCH_PAD // 16
NV_REAL = -(-NCH // 16)
NW = 32
QPW = Q // NW
NSEL = 48
NCAND = 96
NCV = NCAND // 16
BIG = 1e30
CUT = 1e29


def _stage_a(obs_ref, obs_sq_ref, keys_ref, key_sq_ref, w1_ref, b1_ref,
             w2_ref, b2_ref, w3_ref, b3_ref, dists_ref, cmin_ref, qnet_ref):
    i = pl.program_id(0)
    dot = lax.dot_general(obs_ref[...], keys_ref[...],
                          (((1,), (1,)), ((), ())),
                          preferred_element_type=jnp.float32)
    dblk = obs_sq_ref[...] - 2.0 * dot + key_sq_ref[...]
    dists_ref[...] = dblk.reshape(Q, KB // CH, CH)
    cmin_ref[...] = jnp.min(dblk.reshape(Q, KB // CH, CH), axis=2).reshape(
        1, Q, KB // CH)

    @pl.when(i == 0)
    def _mlp():
        h = jnp.maximum(
            lax.dot_general(obs_ref[...], w1_ref[...],
                            (((1,), (1,)), ((), ())),
                            preferred_element_type=jnp.float32) + b1_ref[...],
            0.0)
        h = jnp.maximum(
            lax.dot_general(h, w2_ref[...], (((1,), (1,)), ((), ())),
                            preferred_element_type=jnp.float32) + b2_ref[...],
            0.0)
        qnet_ref[...] = lax.dot_general(
            h, w3_ref[...], (((1,), (1,)), ((), ())),
            preferred_element_type=jnp.float32) + b3_ref[...]


def _splat_f(x):
    return jnp.full((16,), x, jnp.float32)


def _splat_i(x):
    return jnp.full((16,), x, jnp.int32)


_LANE15 = functools.partial(jnp.full, (16,), 15, jnp.int32)


_GDN = lax.GatherDimensionNumbers(
    offset_dims=(), collapsed_slice_dims=(0,), start_index_map=(0,))


def _lane_gather(v, idx):
    return lax.gather(v, idx[:, None], _GDN, (1,),
                      mode=lax.GatherScatterMode.PROMISE_IN_BOUNDS)


def _hmax(v):
    return _lane_gather(plsc.cummax(v), _LANE15())


def _hmin(v):
    return -_hmax(-v)


def _sc_body(dists_hbm, cmin_hbm, qnet_hbm, values_hbm, act_hbm,
             cmins_v, qnet_v, tbuf_v, chsel_v, chunks_v, cd_v, cp_v, sel_v,
             vrows_v, act_v, sem_c0, sem_c1, sem_v0, sem_v1):
    nc = 2
    wid = lax.axis_index("s") * nc + lax.axis_index("c")
    q0 = wid * QPW
    pltpu.sync_copy(cmin_hbm.at[pl.ds(q0, QPW)], cmins_v)
    pltpu.sync_copy(qnet_hbm.at[pl.ds(q0, QPW)], qnet_v)
    iota = lax.iota(jnp.int32, 16)
    sem_c = [sem_c0, sem_c1]
    sem_v = [sem_v0, sem_v1]

    def pass1(qi, carry):
        q = q0 + qi
        mn_a = cmins_v[qi, pl.ds(0, 16)]
        for j in range(1, NV // 2):
            mn_a = jnp.minimum(mn_a, cmins_v[qi, pl.ds(j * 16, 16)])
        mn_b = cmins_v[qi, pl.ds(NV // 2 * 16, 16)]
        for j in range(NV // 2 + 1, NV):
            mn_b = jnp.minimum(mn_b, cmins_v[qi, pl.ds(j * 16, 16)])
        lo0 = _hmin(jnp.minimum(mn_a, mn_b)) - 1.0
        hi0 = _hmax(jnp.maximum(mn_a, mn_b))

        def bis1(_, lohi):
            lo, hi = lohi
            mid = (lo + hi) * 0.5
            cnt = _splat_i(0)
            for j in range(NV):
                m = cmins_v[qi, pl.ds(j * 16, 16)] <= mid
                cnt = cnt + plsc.all_reduce_population_count(m)
            ge = cnt >= K_NN
            return jnp.where(ge, lo, mid), jnp.where(ge, mid, hi)

        _, t = lax.fori_loop(0, 12, bis1, (lo0, hi0))
        plsc.store_scatter(tbuf_v, [_splat_i(qi)], t, mask=iota == 0)

        padrow = _splat_i(q * NCH + (NCH - 1))
        for j in range(NSEL // 16):
            chsel_v[qi, pl.ds(j * 16, 16)] = padrow
        coff = _splat_i(0)
        for j in range(NV_REAL):
            v = cmins_v[qi, pl.ds(j * 16, 16)]
            m = v <= t
            pf = plsc.cumsum(jnp.where(m, 1, 0))
            pos = coff + pf - 1
            m = jnp.logical_and(m, pos < NSEL)
            rowid = _splat_i(q * NCH + j * 16) + iota
            plsc.store_scatter(chsel_v, [_splat_i(qi), pos], rowid, mask=m)
            coff = coff + plsc.all_reduce_population_count(m)
        return carry

    with jax.named_scope("sc_pass1"):
        lax.fori_loop(0, QPW, pass1, 0)

    pltpu.async_copy(dists_hbm.at[chsel_v.at[0]], chunks_v.at[0], sem_c[0])

    def p2a_one(qi, b):
        q = q0 + qi

        @pl.when(qi + 1 < QPW)
        def _prefetch():
            pltpu.async_copy(dists_hbm.at[chsel_v.at[qi + 1]],
                             chunks_v.at[1 - b], sem_c[1 - b])

        with jax.named_scope("sc_wait"):
            pltpu.make_async_copy(dists_hbm.at[chsel_v.at[qi]],
                                  chunks_v.at[b], sem_c[b]).wait()
        t = plsc.load_gather(tbuf_v, [_splat_i(qi)])
        for j in range(NCV):
            cd_v[pl.ds(j * 16, 16)] = _splat_f(BIG)

        def chunk_body(j, eoff):
            for e in range(CH // 16):
                v = chunks_v[b, j, pl.ds(e * 16, 16)]
                cnt = plsc.all_reduce_population_count(v <= t)
                lpos = _splat_i(j * CH + e * 16) + iota
                sv, sp = plsc.sort_key_val(v, lpos)
                pos = jnp.minimum(eoff + iota, NCAND - 1)
                plsc.store_scatter(cd_v, [pos], sv)
                plsc.store_scatter(cp_v, [pos], sp)
                eoff = eoff + cnt
            return eoff

        with jax.named_scope("sc_filter"):
            lax.fori_loop(0, NSEL, chunk_body, _splat_i(0))

        cmn = cd_v[pl.ds(0, 16)]
        for j in range(1, NCV):
            cmn = jnp.minimum(cmn, cd_v[pl.ds(j * 16, 16)])
        clo0 = _hmin(cmn) - 1.0

        def bis2(_, lohi):
            lo2, hi2 = lohi
            mid = (lo2 + hi2) * 0.5
            cnt = _splat_i(0)
            for j in range(NCV):
                m = cd_v[pl.ds(j * 16, 16)] <= mid
                cnt = cnt + plsc.all_reduce_population_count(m)
            ge = cnt >= K_NN
            return jnp.where(ge, lo2, mid), jnp.where(ge, mid, hi2)

        with jax.named_scope("sc_bis2"):
            clo, chi = lax.fori_loop(0, 40, bis2, (clo0, t))

        c1 = _splat_i(0)
        for j in range(NCV):
            c1 = c1 + plsc.all_reduce_population_count(
                cd_v[pl.ds(j * 16, 16)] <= clo)
        need = _splat_i(K_NN) - c1
        c2run = _splat_i(0)
        soff = _splat_i(0)
        for j in range(NCV):
            d = cd_v[pl.ds(j * 16, 16)]
            p = cp_v[pl.ds(j * 16, 16)]
            m1 = d <= clo
            m2 = jnp.logical_and(d <= chi, jnp.logical_not(m1))
            pf2 = plsc.cumsum(jnp.where(m2, 1, 0))
            sel2 = jnp.logical_and(m2, (pf2 + c2run) <= need)
            c2run = c2run + plsc.all_reduce_population_count(m2)
            selm = jnp.logical_or(m1, sel2)
            rowv = plsc.load_gather(
                chsel_v.at[qi], [lax.shift_right_logical(p, 7)], mask=selm)
            gkey = (rowv - q * NCH) * CH + jnp.bitwise_and(p, CH - 1)
            pfs = plsc.cumsum(jnp.where(selm, 1, 0))
            plsc.store_scatter(sel_v, [_splat_i(qi), soff + pfs - 1], gkey,
                               mask=selm)
            soff = soff + plsc.all_reduce_population_count(selm)

    def pass2a(i, carry):
        p2a_one(2 * i, 0)
        p2a_one(2 * i + 1, 1)
        return carry

    with jax.named_scope("sc_pass2a"):
        lax.fori_loop(0, QPW // 2, pass2a, 0)

    pltpu.async_copy(values_hbm.at[sel_v.at[0]], vrows_v.at[0], sem_v[0])

    def p2b_one(qi, b):
        @pl.when(qi + 1 < QPW)
        def _prefetch():
            pltpu.async_copy(values_hbm.at[sel_v.at[qi + 1]],
                             vrows_v.at[1 - b], sem_v[1 - b])

        pltpu.make_async_copy(values_hbm.at[sel_v.at[qi]],
                              vrows_v.at[b], sem_v[b]).wait()
        acc = vrows_v[b, 0, pl.ds(0, 16)]
        for r in range(1, K_NN):
            acc = acc + vrows_v[b, r, pl.ds(0, 16)]
        qvec = acc * (1.0 / K_NN) + qnet_v[qi, pl.ds(0, 16)]
        amax = _hmax(qvec)
        aidx = jnp.where(qvec == amax, iota, 16)
        act = -_hmax(-aidx)
        plsc.store_scatter(act_v, [_splat_i(qi)], act, mask=iota == 0)

    def pass2b(i, carry):
        p2b_one(2 * i, 0)
        p2b_one(2 * i + 1, 1)
        return carry

    with jax.named_scope("sc_pass2b"):
        lax.fori_loop(0, QPW // 2, pass2b, 0)
    pltpu.sync_copy(act_v, act_hbm.at[pl.ds(q0, QPW)])


def _make_sc_kernel():
    return functools.partial(
        pl.kernel,
        out_type=jax.ShapeDtypeStruct((Q,), jnp.int32),
        mesh=plsc.VectorSubcoreMesh(core_axis_name="c", subcore_axis_name="s",
                                    num_cores=2, num_subcores=16),
        compiler_params=pltpu.CompilerParams(needs_layout_passes=False,
                                             use_tc_tiling_on_sc=True),
        scratch_types=[
            pltpu.VMEM((QPW, NCH_PAD), jnp.float32),
            pltpu.VMEM((QPW, 128), jnp.float32),
            pltpu.VMEM((QPW,), jnp.float32),
            pltpu.VMEM((QPW, NSEL), jnp.int32),
            pltpu.VMEM((2, NSEL, CH), jnp.float32),
            pltpu.VMEM((NCAND,), jnp.float32),
            pltpu.VMEM((NCAND,), jnp.int32),
            pltpu.VMEM((QPW, K_NN), jnp.int32),
            pltpu.VMEM((2, K_NN, 128), jnp.float32),
            pltpu.VMEM((QPW,), jnp.int32),
            pltpu.SemaphoreType.DMA,
            pltpu.SemaphoreType.DMA,
            pltpu.SemaphoreType.DMA,
            pltpu.SemaphoreType.DMA,
        ],
    )(_sc_body)


def kernel(observation, keys, values, W1, b1, W2, b2, W3, b3):
    pad = CAP_PAD - CAP
    keys_p = jnp.concatenate([keys, jnp.zeros((pad, D), jnp.float32)], axis=0)
    key_sq = jnp.sum(keys * keys, axis=-1)
    key_sq_p = jnp.concatenate([key_sq, jnp.full((pad,), BIG, jnp.float32)])
    obs_sq = jnp.sum(observation * observation, axis=-1, keepdims=True)
    w3_p = jnp.concatenate([W3, jnp.zeros((128 - A, H), jnp.float32)], axis=0)
    b3_p = jnp.concatenate([b3, jnp.full((128 - A,), -BIG, jnp.float32)])
    values_p = jnp.concatenate(
        [values, jnp.zeros((CAP, 128 - A), jnp.float32)], axis=1)

    full = lambda s: pl.BlockSpec(s, lambda i: tuple(0 for _ in s))
    dists, cmin3, qnet = pl.pallas_call(
        _stage_a,
        grid=(N_BLK,),
        in_specs=[
            full((Q, D)),
            full((Q, 1)),
            pl.BlockSpec((KB, D), lambda i: (i, 0)),
            pl.BlockSpec((1, KB), lambda i: (0, i)),
            full((H, D)),
            full((1, H)),
            full((H, H)),
            full((1, H)),
            full((128, H)),
            full((1, 128)),
        ],
        out_specs=[
            pl.BlockSpec((Q, KB // CH, CH), lambda i: (0, i, 0)),
            pl.BlockSpec((1, Q, KB // CH), lambda i: (i, 0, 0)),
            full((Q, 128)),
        ],
        out_shape=[
            jax.ShapeDtypeStruct((Q, NCH, CH), jnp.float32),
            jax.ShapeDtypeStruct((N_BLK, Q, KB // CH), jnp.float32),
            jax.ShapeDtypeStruct((Q, 128), jnp.float32),
        ],
    )(observation, obs_sq, keys_p, key_sq_p.reshape(1, CAP_PAD),
      W1, b1.reshape(1, H), W2, b2.reshape(1, H), w3_p, b3_p.reshape(1, 128))

    cmin = jnp.transpose(cmin3, (1, 0, 2)).reshape(Q, NCH)
    cmin = jnp.concatenate(
        [cmin, jnp.full((Q, NCH_PAD - NCH), BIG, jnp.float32)], axis=1)
    dists_flat = dists.reshape(Q * NCH, CH)

    return _make_sc_kernel()(dists_flat, cmin, qnet, values_p)


# speedup vs baseline: 1.7366x; 1.0003x over previous
import functools

import jax
import jax.numpy as jnp
from jax import lax
from jax.experimental import pallas as pl
from jax.experimental.pallas import tpu as pltpu
from jax.experimental.pallas import tpu_sc as plsc

Q, D, CAP, A, K_NN, H = 1024, 128, 100000, 8, 32, 64
KB = 2048
CAP_PAD = 100352
N_BLK = CAP_PAD // KB
CH = 128
NCH = CAP_PAD // CH
NCH_PAD = 896
NV = NCH_PAD // 16
NV_REAL = -(-NCH // 16)
NW = 32
QPW = Q // NW
NSEL = 48
NCAND = 96
NCV = NCAND // 16
BIG = 1e30
CUT = 1e29


def _stage_a(obs_ref, obs_sq_ref, keys_ref, key_sq_ref, w1_ref, b1_ref,
             w2_ref, b2_ref, w3_ref, b3_ref, dists_ref, cmin_ref, qnet_ref):
    i = pl.program_id(0)
    dot = lax.dot_general(obs_ref[...], keys_ref[...],
                          (((1,), (1,)), ((), ())),
                          preferred_element_type=jnp.float32)
    dblk = obs_sq_ref[...] - 2.0 * dot + key_sq_ref[...]
    dists_ref[...] = dblk.reshape(Q, KB // CH, CH)
    cmin_ref[...] = jnp.min(dblk.reshape(Q, KB // CH, CH), axis=2).reshape(
        1, Q, KB // CH)

    @pl.when(i == 0)
    def _mlp():
        h = jnp.maximum(
            lax.dot_general(obs_ref[...], w1_ref[...],
                            (((1,), (1,)), ((), ())),
                            preferred_element_type=jnp.float32) + b1_ref[...],
            0.0)
        h = jnp.maximum(
            lax.dot_general(h, w2_ref[...], (((1,), (1,)), ((), ())),
                            preferred_element_type=jnp.float32) + b2_ref[...],
            0.0)
        qnet_ref[...] = lax.dot_general(
            h, w3_ref[...], (((1,), (1,)), ((), ())),
            preferred_element_type=jnp.float32) + b3_ref[...]


def _splat_f(x):
    return jnp.full((16,), x, jnp.float32)


def _splat_i(x):
    return jnp.full((16,), x, jnp.int32)


_LANE15 = functools.partial(jnp.full, (16,), 15, jnp.int32)


_GDN = lax.GatherDimensionNumbers(
    offset_dims=(), collapsed_slice_dims=(0,), start_index_map=(0,))


def _lane_gather(v, idx):
    return lax.gather(v, idx[:, None], _GDN, (1,),
                      mode=lax.GatherScatterMode.PROMISE_IN_BOUNDS)


def _hmax(v):
    return _lane_gather(plsc.cummax(v), _LANE15())


def _hmin(v):
    return -_hmax(-v)


def _sc_body(dists_hbm, cmin_hbm, qnet_hbm, values_hbm, act_hbm,
             cmins_v, qnet_v, tbuf_v, chsel_v, chunks_v, cd_v, cp_v, sel_v,
             vrows_v, act_v, sem_c0, sem_c1, sem_v0, sem_v1):
    nc = 2
    wid = lax.axis_index("s") * nc + lax.axis_index("c")
    q0 = wid * QPW
    pltpu.sync_copy(cmin_hbm.at[pl.ds(q0, QPW)], cmins_v)
    pltpu.sync_copy(qnet_hbm.at[pl.ds(q0, QPW)], qnet_v)
    iota = lax.iota(jnp.int32, 16)
    sem_c = [sem_c0, sem_c1]
    sem_v = [sem_v0, sem_v1]

    def pass1(qi, carry):
        q = q0 + qi
        mn_a = cmins_v[qi, pl.ds(0, 16)]
        for j in range(1, NV // 2):
            mn_a = jnp.minimum(mn_a, cmins_v[qi, pl.ds(j * 16, 16)])
        mn_b = cmins_v[qi, pl.ds(NV // 2 * 16, 16)]
        for j in range(NV // 2 + 1, NV):
            mn_b = jnp.minimum(mn_b, cmins_v[qi, pl.ds(j * 16, 16)])
        lo0 = _hmin(jnp.minimum(mn_a, mn_b)) - 1.0
        hi0 = _hmax(jnp.maximum(mn_a, mn_b))

        def bis1(_, lohi):
            lo, hi = lohi
            mid = (lo + hi) * 0.5
            cnt = _splat_i(0)
            for j in range(NV):
                m = cmins_v[qi, pl.ds(j * 16, 16)] <= mid
                cnt = cnt + plsc.all_reduce_population_count(m)
            ge = cnt >= K_NN
            return jnp.where(ge, lo, mid), jnp.where(ge, mid, hi)

        _, t = lax.fori_loop(0, 12, bis1, (lo0, hi0))
        plsc.store_scatter(tbuf_v, [_splat_i(qi)], t, mask=iota == 0)

        padrow = _splat_i(q * NCH + (NCH - 1))
        for j in range(NSEL // 16):
            chsel_v[qi, pl.ds(j * 16, 16)] = padrow
        coff = _splat_i(0)
        for j in range(NV_REAL):
            v = cmins_v[qi, pl.ds(j * 16, 16)]
            m = v <= t
            pf = plsc.cumsum(jnp.where(m, 1, 0))
            pos = coff + pf - 1
            m = jnp.logical_and(m, pos < NSEL)
            rowid = _splat_i(q * NCH + j * 16) + iota
            plsc.store_scatter(chsel_v, [_splat_i(qi), pos], rowid, mask=m)
            coff = coff + plsc.all_reduce_population_count(m)
        return carry

    lax.fori_loop(0, QPW, pass1, 0)

    pltpu.async_copy(dists_hbm.at[chsel_v.at[0]], chunks_v.at[0], sem_c[0])

    def p2a_one(qi, b):
        q = q0 + qi

        @pl.when(qi + 1 < QPW)
        def _prefetch():
            pltpu.async_copy(dists_hbm.at[chsel_v.at[qi + 1]],
                             chunks_v.at[1 - b], sem_c[1 - b])

        pltpu.make_async_copy(dists_hbm.at[chsel_v.at[qi]],
                              chunks_v.at[b], sem_c[b]).wait()
        t = plsc.load_gather(tbuf_v, [_splat_i(qi)])
        for j in range(NCV):
            cd_v[pl.ds(j * 16, 16)] = _splat_f(BIG)

        def chunk_body(j, eoff):
            for e in range(CH // 16):
                v = chunks_v[b, j, pl.ds(e * 16, 16)]
                cnt = plsc.all_reduce_population_count(v <= t)
                lpos = _splat_i(j * CH + e * 16) + iota
                sv, sp = plsc.sort_key_val(v, lpos)
                pos = jnp.minimum(eoff + iota, NCAND - 1)
                plsc.store_scatter(cd_v, [pos], sv)
                plsc.store_scatter(cp_v, [pos], sp)
                eoff = eoff + cnt
            return eoff

        lax.fori_loop(0, NSEL, chunk_body, _splat_i(0))

        cmn = cd_v[pl.ds(0, 16)]
        for j in range(1, NCV):
            cmn = jnp.minimum(cmn, cd_v[pl.ds(j * 16, 16)])
        clo0 = _hmin(cmn) - 1.0

        def bis2(_, lohi):
            lo2, hi2 = lohi
            mid = (lo2 + hi2) * 0.5
            cnt = _splat_i(0)
            for j in range(NCV):
                m = cd_v[pl.ds(j * 16, 16)] <= mid
                cnt = cnt + plsc.all_reduce_population_count(m)
            ge = cnt >= K_NN
            return jnp.where(ge, lo2, mid), jnp.where(ge, mid, hi2)

        clo, chi = lax.fori_loop(0, 40, bis2, (clo0, t))

        c1 = _splat_i(0)
        for j in range(NCV):
            c1 = c1 + plsc.all_reduce_population_count(
                cd_v[pl.ds(j * 16, 16)] <= clo)
        need = _splat_i(K_NN) - c1
        c2run = _splat_i(0)
        soff = _splat_i(0)
        for j in range(NCV):
            d = cd_v[pl.ds(j * 16, 16)]
            p = cp_v[pl.ds(j * 16, 16)]
            m1 = d <= clo
            m2 = jnp.logical_and(d <= chi, jnp.logical_not(m1))
            pf2 = plsc.cumsum(jnp.where(m2, 1, 0))
            sel2 = jnp.logical_and(m2, (pf2 + c2run) <= need)
            c2run = c2run + plsc.all_reduce_population_count(m2)
            selm = jnp.logical_or(m1, sel2)
            rowv = plsc.load_gather(
                chsel_v.at[qi], [lax.shift_right_logical(p, 7)], mask=selm)
            gkey = (rowv - q * NCH) * CH + jnp.bitwise_and(p, CH - 1)
            pfs = plsc.cumsum(jnp.where(selm, 1, 0))
            plsc.store_scatter(sel_v, [_splat_i(qi), soff + pfs - 1], gkey,
                               mask=selm)
            soff = soff + plsc.all_reduce_population_count(selm)

    def pass2a(i, carry):
        p2a_one(2 * i, 0)
        p2a_one(2 * i + 1, 1)
        return carry

    lax.fori_loop(0, QPW // 2, pass2a, 0)

    pltpu.async_copy(values_hbm.at[sel_v.at[0]], vrows_v.at[0], sem_v[0])

    def p2b_one(qi, b):
        @pl.when(qi + 1 < QPW)
        def _prefetch():
            pltpu.async_copy(values_hbm.at[sel_v.at[qi + 1]],
                             vrows_v.at[1 - b], sem_v[1 - b])

        pltpu.make_async_copy(values_hbm.at[sel_v.at[qi]],
                              vrows_v.at[b], sem_v[b]).wait()
        acc = vrows_v[b, 0, pl.ds(0, 16)]
        for r in range(1, K_NN):
            acc = acc + vrows_v[b, r, pl.ds(0, 16)]
        qvec = acc * (1.0 / K_NN) + qnet_v[qi, pl.ds(0, 16)]
        amax = _hmax(qvec)
        aidx = jnp.where(qvec == amax, iota, 16)
        act = -_hmax(-aidx)
        plsc.store_scatter(act_v, [_splat_i(qi)], act, mask=iota == 0)

    def pass2b(i, carry):
        p2b_one(2 * i, 0)
        p2b_one(2 * i + 1, 1)
        return carry

    lax.fori_loop(0, QPW // 2, pass2b, 0)
    pltpu.sync_copy(act_v, act_hbm.at[pl.ds(q0, QPW)])


def _make_sc_kernel():
    return functools.partial(
        pl.kernel,
        out_type=jax.ShapeDtypeStruct((Q,), jnp.int32),
        mesh=plsc.VectorSubcoreMesh(core_axis_name="c", subcore_axis_name="s",
                                    num_cores=2, num_subcores=16),
        compiler_params=pltpu.CompilerParams(needs_layout_passes=False,
                                             use_tc_tiling_on_sc=True),
        scratch_types=[
            pltpu.VMEM((QPW, NCH_PAD), jnp.float32),
            pltpu.VMEM((QPW, 128), jnp.float32),
            pltpu.VMEM((QPW,), jnp.float32),
            pltpu.VMEM((QPW, NSEL), jnp.int32),
            pltpu.VMEM((2, NSEL, CH), jnp.float32),
            pltpu.VMEM((NCAND,), jnp.float32),
            pltpu.VMEM((NCAND,), jnp.int32),
            pltpu.VMEM((QPW, K_NN), jnp.int32),
            pltpu.VMEM((2, K_NN, 128), jnp.float32),
            pltpu.VMEM((QPW,), jnp.int32),
            pltpu.SemaphoreType.DMA,
            pltpu.SemaphoreType.DMA,
            pltpu.SemaphoreType.DMA,
            pltpu.SemaphoreType.DMA,
        ],
    )(_sc_body)


def kernel(observation, keys, values, W1, b1, W2, b2, W3, b3):
    pad = CAP_PAD - CAP
    keys_p = jnp.concatenate([keys, jnp.zeros((pad, D), jnp.float32)], axis=0)
    key_sq = jnp.sum(keys * keys, axis=-1)
    key_sq_p = jnp.concatenate([key_sq, jnp.full((pad,), BIG, jnp.float32)])
    obs_sq = jnp.sum(observation * observation, axis=-1, keepdims=True)
    w3_p = jnp.concatenate([W3, jnp.zeros((128 - A, H), jnp.float32)], axis=0)
    b3_p = jnp.concatenate([b3, jnp.full((128 - A,), -BIG, jnp.float32)])
    values_p = jnp.concatenate(
        [values, jnp.zeros((CAP, 128 - A), jnp.float32)], axis=1)

    full = lambda s: pl.BlockSpec(s, lambda i: tuple(0 for _ in s))
    dists, cmin3, qnet = pl.pallas_call(
        _stage_a,
        grid=(N_BLK,),
        in_specs=[
            full((Q, D)),
            full((Q, 1)),
            pl.BlockSpec((KB, D), lambda i: (i, 0)),
            pl.BlockSpec((1, KB), lambda i: (0, i)),
            full((H, D)),
            full((1, H)),
            full((H, H)),
            full((1, H)),
            full((128, H)),
            full((1, 128)),
        ],
        out_specs=[
            pl.BlockSpec((Q, KB // CH, CH), lambda i: (0, i, 0)),
            pl.BlockSpec((1, Q, KB // CH), lambda i: (i, 0, 0)),
            full((Q, 128)),
        ],
        out_shape=[
            jax.ShapeDtypeStruct((Q, NCH, CH), jnp.float32),
            jax.ShapeDtypeStruct((N_BLK, Q, KB // CH), jnp.float32),
            jax.ShapeDtypeStruct((Q, 128), jnp.float32),
        ],
    )(observation, obs_sq, keys_p, key_sq_p.reshape(1, CAP_PAD),
      W1, b1.reshape(1, H), W2, b2.reshape(1, H), w3_p, b3_p.reshape(1, 128))

    cmin = jnp.transpose(cmin3, (1, 0, 2)).reshape(Q, NCH)
    cmin = jnp.concatenate(
        [cmin, jnp.full((Q, NCH_PAD - NCH), BIG, jnp.float32)], axis=1)
    dists_flat = dists.reshape(Q * NCH, CH)

    return _make_sc_kernel()(dists_flat, cmin, qnet, values_p)
